# Initial kernel scaffold; baseline (speedup 1.0000x reference)
import jax
import jax.numpy as jnp
from jax.experimental import pallas as pl


def kernel(inputs, tables):
    raise NotImplementedError("write your pallas kernel here")


# speedup vs baseline: 7.0348x; 7.0348x over previous
import functools

import jax
import jax.numpy as jnp
from jax import lax
from jax.experimental import pallas as pl
from jax.experimental.pallas import tpu as pltpu
from jax.experimental.pallas import tpu_sc as plsc

F = 26
B = 4096
H = 20
V = 100000
D = 32

NW = 32
SLOTS_PER_TILE = (B * F) // NW
C = 128
CHUNKS = SLOTS_PER_TILE // C


def _make_sc_kernel():
    info = plsc.get_sparse_core_info()
    nc = info.num_cores
    mesh = plsc.VectorSubcoreMesh(core_axis_name="c", subcore_axis_name="s")

    @functools.partial(
        pl.kernel,
        mesh=mesh,
        compiler_params=pltpu.CompilerParams(use_tc_tiling_on_sc=False),
        out_type=jax.ShapeDtypeStruct((B * F, D), jnp.float32),
        scratch_types=[
            pltpu.VMEM((C * H,), jnp.int32),
            pltpu.VMEM((C * H, D), jnp.float32),
            pltpu.VMEM((C, D), jnp.float32),
            pltpu.SemaphoreType.DMA,
        ],
    )
    def k(table_hbm, idx_hbm, out_hbm, idx_v, rows_v, acc_v, sem):
        wid = lax.axis_index("s") * nc + lax.axis_index("c")
        tile_base = wid * SLOTS_PER_TILE

        def chunk_body(ci, carry):
            slot_base = tile_base + ci * C
            pltpu.sync_copy(idx_hbm.at[pl.ds(slot_base * H, C * H)], idx_v)
            pltpu.async_copy(table_hbm.at[idx_v], rows_v, sem).wait()

            def slot_body(s, c2):
                base = s * H
                a0 = rows_v[base, pl.ds(0, 16)]
                a1 = rows_v[base, pl.ds(16, 16)]
                for l in range(1, H):
                    a0 = a0 + rows_v[base + l, pl.ds(0, 16)]
                    a1 = a1 + rows_v[base + l, pl.ds(16, 16)]
                acc_v[s, pl.ds(0, 16)] = a0
                acc_v[s, pl.ds(16, 16)] = a1
                return c2

            lax.fori_loop(0, C, slot_body, 0)
            pltpu.sync_copy(acc_v, out_hbm.at[pl.ds(slot_base, C)])
            return carry

        lax.fori_loop(0, CHUNKS, chunk_body, 0)

    return k


_sc_kernel = _make_sc_kernel()


@jax.jit
def kernel(inputs, tables):
    idx = inputs.astype(jnp.int32) + (jnp.arange(F, dtype=jnp.int32) * V)[:, None, None]
    idx_flat = jnp.transpose(idx, (1, 0, 2)).reshape(B * F * H)
    tables_flat = tables.reshape(F * V, D)
    out = _sc_kernel(tables_flat, idx_flat)
    return out.reshape(B, F, D)


# speedup vs baseline: 7.6667x; 1.0898x over previous
import functools

import jax
import jax.numpy as jnp
from jax import lax
from jax.experimental import pallas as pl
from jax.experimental.pallas import tpu as pltpu
from jax.experimental.pallas import tpu_sc as plsc

F = 26
B = 4096
H = 20
V = 100000
D = 32

NW = 32
SLOTS_PER_TILE = (B * F) // NW
C = 1664
CHUNKS = SLOTS_PER_TILE // C


def _make_sc_kernel():
    info = plsc.get_sparse_core_info()
    nc = info.num_cores
    mesh = plsc.VectorSubcoreMesh(core_axis_name="c", subcore_axis_name="s")

    @functools.partial(
        pl.kernel,
        mesh=mesh,
        compiler_params=pltpu.CompilerParams(use_tc_tiling_on_sc=False),
        out_type=jax.ShapeDtypeStruct((B * F, D), jnp.float32),
        scratch_types=[
            pltpu.VMEM((H, C), jnp.int32),
            pltpu.VMEM((C, D), jnp.float32),
            pltpu.SemaphoreType.DMA,
        ],
    )
    def k(table_hbm, idx_hbm, zeros_hbm, out_hbm, idx_v, acc_v, sem):
        wid = lax.axis_index("s") * nc + lax.axis_index("c")
        tile_base = wid * SLOTS_PER_TILE

        def chunk_body(ci, carry):
            slot_base = tile_base + ci * C
            pltpu.sync_copy(idx_hbm.at[:, pl.ds(slot_base, C)], idx_v)
            pltpu.sync_copy(zeros_hbm, acc_v)
            for l in range(H):
                pltpu.async_copy(table_hbm.at[idx_v.at[l]], acc_v, sem, add=True)
            for l in range(H):
                pltpu.make_async_copy(table_hbm.at[idx_v.at[l]], acc_v, sem).wait()
            pltpu.sync_copy(acc_v, out_hbm.at[pl.ds(slot_base, C)])
            return carry

        lax.fori_loop(0, CHUNKS, chunk_body, 0)

    return k


_sc_kernel = _make_sc_kernel()


@jax.jit
def kernel(inputs, tables):
    idx = inputs.astype(jnp.int32) + (jnp.arange(F, dtype=jnp.int32) * V)[:, None, None]
    idx_lt = jnp.transpose(idx, (2, 1, 0)).reshape(H, B * F)
    tables_flat = tables.reshape(F * V, D)
    zeros = jnp.zeros((C, D), jnp.float32)
    out = _sc_kernel(tables_flat, idx_lt, zeros)
    return out.reshape(B, F, D)
